# Initial kernel scaffold; baseline (speedup 1.0000x reference)
import jax
import jax.numpy as jnp
from jax.experimental import pallas as pl


def kernel(x, pos, batch, W1, b1, W2, b2):
    raise NotImplementedError("write your pallas kernel here")


# speedup vs baseline: 6.0673x; 6.0673x over previous
import functools

import jax
import jax.numpy as jnp
from jax import lax
from jax.experimental import pallas as pl
from jax.experimental.pallas import tpu as pltpu
from jax.experimental.pallas import tpu_sc as plsc

N = 32768
Q = 8192
K = 64
D = 64
R2 = 0.01

FPS_R = 8
FPS_C = N // FPS_R

BQ = 32
NEG_BIG = -1e9
INF = float("inf")


def _fps_body(px, py, pz, sel_ref, scratch_mind):
    del scratch_mind
    pxv = px[...]
    pyv = py[...]
    pzv = pz[...]
    flat_iota = lax.broadcasted_iota(jnp.int32, (FPS_R, FPS_C), 0) * FPS_C + \
        lax.broadcasted_iota(jnp.int32, (FPS_R, FPS_C), 1)

    c0 = (flat_iota == 0).astype(jnp.float32)
    cx0 = jnp.sum(pxv * c0)
    cy0 = jnp.sum(pyv * c0)
    cz0 = jnp.sum(pzv * c0)
    dx = pxv - cx0
    dy = pyv - cy0
    dz = pzv - cz0
    mind0 = (dx * dx + dy * dy) + dz * dz

    sel_ref[...] = jnp.zeros((Q // 128, 128), jnp.int32)
    qiota = lax.broadcasted_iota(jnp.int32, (Q // 128, 128), 0) * 128 + \
        lax.broadcasted_iota(jnp.int32, (Q // 128, 128), 1)

    def body(i, mind):
        m = jnp.max(mind)
        cand = jnp.where(mind == m, flat_iota, N)
        nxt = jnp.min(cand)
        cm = (flat_iota == nxt).astype(jnp.float32)
        cx = jnp.sum(pxv * cm)
        cy = jnp.sum(pyv * cm)
        cz = jnp.sum(pzv * cm)
        ddx = pxv - cx
        ddy = pyv - cy
        ddz = pzv - cz
        d = (ddx * ddx + ddy * ddy) + ddz * ddz
        sel_ref[...] = jnp.where(qiota == i, nxt, sel_ref[...])
        return jnp.minimum(mind, d)

    lax.fori_loop(1, Q, body, mind0)


def _fps(pos):
    px = pos[:, 0].reshape(FPS_R, FPS_C)
    py = pos[:, 1].reshape(FPS_R, FPS_C)
    pz = pos[:, 2].reshape(FPS_R, FPS_C)
    sel2d = pl.pallas_call(
        _fps_body,
        out_shape=jax.ShapeDtypeStruct((Q // 128, 128), jnp.int32),
        scratch_shapes=[pltpu.VMEM((FPS_R, FPS_C), jnp.float32)],
    )(px, py, pz)
    return sel2d.reshape(Q)


def _sc_gather(table, idx, d_cols, chunk):
    b_total = idx.shape[0]
    info = plsc.get_sparse_core_info()
    nw = info.num_cores * info.num_subcores
    b_per_w = b_total // nw
    n_chunks = b_per_w // chunk
    mesh = plsc.VectorSubcoreMesh(core_axis_name="c", subcore_axis_name="s")

    @functools.partial(
        pl.kernel, mesh=mesh,
        out_type=jax.ShapeDtypeStruct((b_total, d_cols), jnp.float32),
        scratch_types=[
            pltpu.VMEM((chunk,), jnp.int32),
            pltpu.VMEM((chunk, d_cols), jnp.float32),
            pltpu.SemaphoreType.DMA,
        ],
    )
    def k(table_hbm, idx_hbm, out_hbm, idx_v, rows_v, sem):
        wid = lax.axis_index("s") * info.num_cores + lax.axis_index("c")
        base = wid * b_per_w

        def step(c, _):
            off = base + c * chunk
            pltpu.sync_copy(idx_hbm.at[pl.ds(off, chunk)], idx_v)
            pltpu.async_copy(table_hbm.at[idx_v], rows_v, sem).wait()
            pltpu.sync_copy(rows_v, out_hbm.at[pl.ds(off, chunk)])
            return 0

        lax.fori_loop(0, n_chunks, step, 0)

    return k(table, idx)


def _topk_body(pxr, pyr, pzr, qrow, idx_ref, d2k_ref, d2w):
    qx = qrow[:, 0:1]
    qy = qrow[:, 1:2]
    qz = qrow[:, 2:3]
    dx = pxr[...] - qx
    dy = pyr[...] - qy
    dz = pzr[...] - qz
    d2w[...] = (dx * dx + dy * dy) + dz * dz

    lane_iota = lax.broadcasted_iota(jnp.int32, (BQ, N), 1)
    k_iota = lax.broadcasted_iota(jnp.int32, (BQ, K), 1)

    def body(k, carry):
        idx_acc, d2_acc = carry
        w = d2w[...]
        m = jnp.min(w, axis=1, keepdims=True)
        cand = jnp.where(w == m, lane_iota, N)
        j = jnp.min(cand, axis=1, keepdims=True)
        d2w[...] = jnp.where(lane_iota == j, INF, w)
        idx_acc = jnp.where(k_iota == k, j, idx_acc)
        d2_acc = jnp.where(k_iota == k, m, d2_acc)
        return idx_acc, d2_acc

    idx0 = jnp.zeros((BQ, K), jnp.int32)
    d20 = jnp.zeros((BQ, K), jnp.float32)
    idx_acc, d2_acc = lax.fori_loop(0, K, body, (idx0, d20))
    idx_ref[...] = idx_acc
    d2k_ref[...] = d2_acc


def _topk(pos, qrows):
    pxr = pos[:, 0].reshape(1, N)
    pyr = pos[:, 1].reshape(1, N)
    pzr = pos[:, 2].reshape(1, N)
    grid = Q // BQ
    idx, d2k = pl.pallas_call(
        _topk_body,
        grid=(grid,),
        in_specs=[
            pl.BlockSpec((1, N), lambda i: (0, 0)),
            pl.BlockSpec((1, N), lambda i: (0, 0)),
            pl.BlockSpec((1, N), lambda i: (0, 0)),
            pl.BlockSpec((BQ, 128), lambda i: (i, 0)),
        ],
        out_specs=[
            pl.BlockSpec((BQ, K), lambda i: (i, 0)),
            pl.BlockSpec((BQ, K), lambda i: (i, 0)),
        ],
        out_shape=[
            jax.ShapeDtypeStruct((Q, K), jnp.int32),
            jax.ShapeDtypeStruct((Q, K), jnp.float32),
        ],
        scratch_shapes=[pltpu.VMEM((BQ, N), jnp.float32)],
    )(pxr, pyr, pzr, qrows)
    return idx, d2k


def _gtable_body(x, w1x, b1r, out_ref):
    out_ref[...] = jnp.dot(x[...], w1x[...],
                           preferred_element_type=jnp.float32) + b1r[0:1, :]


def _gtable(x, w1x, b1):
    b1r = jnp.broadcast_to(b1.reshape(1, D), (8, D))
    grid = N // 2048
    return pl.pallas_call(
        _gtable_body,
        grid=(grid,),
        in_specs=[
            pl.BlockSpec((2048, D), lambda i: (i, 0)),
            pl.BlockSpec((D, D), lambda i: (0, 0)),
            pl.BlockSpec((8, D), lambda i: (0, 0)),
        ],
        out_specs=pl.BlockSpec((2048, D), lambda i: (i, 0)),
        out_shape=jax.ShapeDtypeStruct((N, D), jnp.float32),
    )(x, w1x, b1r)


BROW = 8192


def _mlp_body(rows, qd, w1p, w2, b2r, out_ref):
    g = rows[:, 0:64]
    rel8 = rows[:, 64:72] - qd[:, 0:8]
    h1 = g + jnp.dot(rel8, w1p[...], preferred_element_type=jnp.float32)
    h1 = jnp.maximum(h1, 0.0)
    h2 = jnp.dot(h1, w2[...], preferred_element_type=jnp.float32) + b2r[0:1, :]
    h2 = jnp.maximum(h2, 0.0)
    valid = qd[:, 8:9] <= R2
    h2 = jnp.where(valid, h2, NEG_BIG)
    h3 = h2.reshape(BROW // K, K, D)
    out_ref[...] = jnp.max(h3, axis=1)


def _mlp_max(rows, qd, w1p8, w2, b2):
    b2r = jnp.broadcast_to(b2.reshape(1, D), (8, D))
    grid = (Q * K) // BROW
    return pl.pallas_call(
        _mlp_body,
        grid=(grid,),
        in_specs=[
            pl.BlockSpec((BROW, 128), lambda i: (i, 0)),
            pl.BlockSpec((BROW, 16), lambda i: (i, 0)),
            pl.BlockSpec((8, D), lambda i: (0, 0)),
            pl.BlockSpec((D, D), lambda i: (0, 0)),
            pl.BlockSpec((8, D), lambda i: (0, 0)),
        ],
        out_specs=pl.BlockSpec((BROW // K, D), lambda i: (i, 0)),
        out_shape=jax.ShapeDtypeStruct((Q, D), jnp.float32),
    )(rows, qd, w1p8, w2, b2r)


def kernel(x, pos, batch, W1, b1, W2, b2):
    sel = _fps(pos)

    batchf = batch.astype(jnp.float32)
    pos128 = jnp.concatenate(
        [pos, batchf[:, None], jnp.zeros((N, 124), jnp.float32)], axis=1)
    qrows = _sc_gather(pos128, sel, 128, 256)
    qpos = qrows[:, 0:3]
    batch1 = qrows[:, 3].astype(jnp.int32)

    idx, d2k = _topk(pos, qrows)

    w1x = W1[0:D, :]
    g = _gtable(x, w1x, b1)
    table128 = jnp.concatenate([g, pos, jnp.zeros((N, 61), jnp.float32)],
                               axis=1)
    rows = _sc_gather(table128, idx.reshape(Q * K), 128, 512)

    qrep = jnp.broadcast_to(qpos[:, None, :], (Q, K, 3)).reshape(Q * K, 3)
    qd = jnp.concatenate(
        [qrep, jnp.zeros((Q * K, 5), jnp.float32),
         d2k.reshape(Q * K, 1), jnp.zeros((Q * K, 7), jnp.float32)], axis=1)

    w1p8 = jnp.concatenate([W1[D:D + 3, :], jnp.zeros((5, D), jnp.float32)],
                           axis=0)
    x1 = _mlp_max(rows, qd, w1p8, W2, b2)
    return (x1, qpos, batch1)
